# Initial kernel scaffold; baseline (speedup 1.0000x reference)
import jax
import jax.numpy as jnp
from jax.experimental import pallas as pl


def kernel(attr_item, attr_tf_item, attr_lens_item, item_ids, attr_user, attr_tf_user, attr_lens_user, user_ids, pos_targets, pos_lens, neg_targets, neg_lens, attr_table, user_table, item_table, out_user_table, out_item_table):
    raise NotImplementedError("write your pallas kernel here")


# speedup vs baseline: 3.1271x; 3.1271x over previous
import functools

import jax
import jax.numpy as jnp
from jax import lax
from jax.experimental import pallas as pl
from jax.experimental.pallas import tpu as pltpu
from jax.experimental.pallas import tpu_sc as plsc

B = 4096
LA = 50
LP = 20
LN = 200
D = 64
L = 220
GAMMA = 0.5

NC = 2
NS = 16
NW = NC * NS
BPW = B // NW

IW = 384
NA = 128
NT = 240
WW = 128


def _mask_body(pos_l_ref, neg_l_ref, ali_ref, alu_ref, mask_ref, tgt_ref, w_ref):
    iota = lax.broadcasted_iota(jnp.int32, (B, L), 1)
    pos_l = pos_l_ref[...]
    neg_l = neg_l_ref[...]
    mp = jnp.where(iota < pos_l, 1, 0)
    mn = jnp.where((iota - LP) < neg_l, 1, 0)
    m = jnp.where(iota < LP, mp, mn)
    mask_ref[...] = m
    tgt_ref[...] = jnp.where(iota < LP, m, 0)
    iw = lax.broadcasted_iota(jnp.int32, (B, WW), 1)
    lens = jnp.where(iw < 64, ali_ref[...], alu_ref[...])
    j = jnp.where(iw < 64, iw, iw - 64)
    w_ref[...] = jnp.where(
        j < lens, GAMMA / lens.astype(jnp.float32), jnp.float32(0.0))


_mask_call = pl.pallas_call(
    _mask_body,
    out_shape=(
        jax.ShapeDtypeStruct((B, L), jnp.int32),
        jax.ShapeDtypeStruct((B, L), jnp.int32),
        jax.ShapeDtypeStruct((B, WW), jnp.float32),
    ),
)


def _sc_body(idx_hbm, w_hbm, attr_t, user_t, item_t, ou_t, oi_t, out_hbm,
             idx_v, w_v, arows, urow, irow, turows, tirows, logits_v, sem):
    wid = lax.axis_index("s") * NC + lax.axis_index("c")

    def row_body(bi, carry):
        b = wid * BPW + bi
        pltpu.sync_copy(idx_hbm.at[b], idx_v)
        pltpu.sync_copy(w_hbm.at[b], w_v)
        cps = [
            pltpu.async_copy(user_t.at[idx_v.at[pl.ds(0, 1)]], urow, sem),
            pltpu.async_copy(item_t.at[idx_v.at[pl.ds(8, 1)]], irow, sem),
            pltpu.async_copy(attr_t.at[idx_v.at[pl.ds(16, NA)]], arows, sem),
            pltpu.async_copy(ou_t.at[idx_v.at[pl.ds(144, 128)]],
                             turows.at[pl.ds(0, 128)], sem),
            pltpu.async_copy(ou_t.at[idx_v.at[pl.ds(272, 112)]],
                             turows.at[pl.ds(128, 112)], sem),
            pltpu.async_copy(oi_t.at[idx_v.at[pl.ds(144, 128)]],
                             tirows.at[pl.ds(0, 128)], sem),
            pltpu.async_copy(oi_t.at[idx_v.at[pl.ds(272, 112)]],
                             tirows.at[pl.ds(128, 112)], sem),
        ]
        for c in cps:
            c.wait()

        zero = jnp.zeros((16,), jnp.float32)
        accs = (zero,) * 8
        for c in range(4):
            wci = w_v[pl.ds(c * 16, 16)]
            wcu = w_v[pl.ds(64 + c * 16, 16)]

            def attr_body(t16, a, c=c, wci=wci, wcu=wcu):
                j = c * 16 + t16
                sp = jnp.full((16,), t16, jnp.int32)
                wi = jnp.take_along_axis(wci, sp, axis=0,
                                         mode="promise_in_bounds")
                wu = jnp.take_along_axis(wcu, sp, axis=0,
                                         mode="promise_in_bounds")
                out = []
                for k in range(4):
                    out.append(a[k] + arows[j, pl.ds(k * 16, 16)] * wi)
                for k in range(4):
                    out.append(a[4 + k] + arows[j + 64, pl.ds(k * 16, 16)] * wu)
                return tuple(out)

            accs = lax.fori_loop(0, 16, attr_body, accs)
        half = jnp.float32(1.0 - GAMMA)
        io = [half * irow[0, pl.ds(k * 16, 16)] + accs[k] for k in range(4)]
        uo = [half * urow[0, pl.ds(k * 16, 16)] + accs[4 + k] for k in range(4)]

        iota16 = lax.iota(jnp.int32, 16)
        xors = [jnp.bitwise_xor(iota16, jnp.int32(dd)) for dd in (8, 4, 2, 1)]

        def tgt_body(g, carry):
            base = g * 16
            lvec = jnp.zeros((16,), jnp.float32)
            for t16 in range(16):
                t = base + t16
                acc = turows[t, pl.ds(0, 16)] * uo[0]
                for k in range(1, 4):
                    acc = acc + turows[t, pl.ds(k * 16, 16)] * uo[k]
                for k in range(4):
                    acc = acc + tirows[t, pl.ds(k * 16, 16)] * io[k]
                for xi in xors:
                    acc = acc + jnp.take_along_axis(
                        acc, xi, axis=0, mode="promise_in_bounds")
                lvec = jnp.where(iota16 == t16, acc, lvec)
            logits_v[pl.ds(base, 16)] = lvec
            return carry

        lax.fori_loop(0, NT // 16, tgt_body, 0)
        pltpu.sync_copy(logits_v.at[pl.ds(8, L)], out_hbm.at[b])
        return carry

    lax.fori_loop(0, BPW, row_body, 0)


_sc_call = functools.partial(
    pl.kernel,
    out_type=jax.ShapeDtypeStruct((B, L), jnp.float32),
    mesh=plsc.VectorSubcoreMesh(core_axis_name="c", subcore_axis_name="s"),
    compiler_params=pltpu.CompilerParams(use_tc_tiling_on_sc=False),
    scratch_types=[
        pltpu.VMEM((IW,), jnp.int32),
        pltpu.VMEM((WW,), jnp.float32),
        pltpu.VMEM((NA, D), jnp.float32),
        pltpu.VMEM((1, D), jnp.float32),
        pltpu.VMEM((1, D), jnp.float32),
        pltpu.VMEM((NT, D), jnp.float32),
        pltpu.VMEM((NT, D), jnp.float32),
        pltpu.VMEM((NT,), jnp.float32),
        pltpu.SemaphoreType.DMA,
    ],
)(_sc_body)


def kernel(attr_item, attr_tf_item, attr_lens_item, item_ids, attr_user,
           attr_tf_user, attr_lens_user, user_ids, pos_targets, pos_lens,
           neg_targets, neg_lens, attr_table, user_table, item_table,
           out_user_table, out_item_table):
    i32 = jnp.int32
    z = lambda n: jnp.zeros((B, n), i32)
    packed_idx = jnp.concatenate([
        user_ids[:, None].astype(i32), z(7),
        item_ids[:, None].astype(i32), z(7),
        attr_item.astype(i32), z(14),
        attr_user.astype(i32), z(14),
        z(8),
        pos_targets.astype(i32),
        neg_targets.astype(i32), z(12),
    ], axis=1)

    mask_i, new_targets, packed_w = _mask_call(
        pos_lens[:, None].astype(i32), neg_lens[:, None].astype(i32),
        attr_lens_item[:, None].astype(i32),
        attr_lens_user[:, None].astype(i32))

    logits = _sc_call(packed_idx, packed_w, attr_table, user_table,
                      item_table, out_user_table, out_item_table)
    return (logits, mask_i.astype(jnp.bool_), new_targets)
